# baseline (speedup 1.0000x reference)
import functools

import jax
import jax.numpy as jnp
from jax import lax
from jax.experimental import pallas as pl
from jax.experimental.pallas import tpu as pltpu
from jax.experimental.pallas import tpu_sc as plsc

C = 128
N = 10000
E = 320000

NC = 2
NS = 16
L = 16
NW = NC * NS
CPT = C // NW
ECH = 1600
NCHUNK = E // ECH
STEPS = ECH // L
SENTINEL = 3.0e38


def _fc1_bn_relu_body(x_ref, w_ref, b_ref, g_ref, bb_ref, o_ref):
    z = jnp.dot(w_ref[...], x_ref[...], preferred_element_type=jnp.float32)
    z = z + b_ref[...]
    m = jnp.mean(z, axis=1, keepdims=True)
    v = jnp.mean((z - m) ** 2, axis=1, keepdims=True)
    h = g_ref[...] * (z - m) * lax.rsqrt(v + 1e-5) + bb_ref[...]
    o_ref[...] = jnp.maximum(h, 0.0)


def _stage_a(x2d, w, b, g, bb):
    return pl.pallas_call(
        _fc1_bn_relu_body,
        out_shape=jax.ShapeDtypeStruct((C, N), jnp.float32),
    )(x2d, w, b, g, bb)


def _segmin_body(h_hbm, src_hbm, dst_hbm, out_hbm, *refs):
    hlocs = refs[0:CPT]
    acc_par = (refs[CPT:2 * CPT], refs[2 * CPT:3 * CPT])
    sv0, dv0, sv1, dv1 = refs[3 * CPT:3 * CPT + 4]
    sem_s0, sem_d0, sem_s1, sem_d1 = refs[3 * CPT + 4:3 * CPT + 8]

    cid = lax.axis_index("c")
    sid = lax.axis_index("s")
    wid = sid * NC + cid
    c0 = wid * CPT

    pltpu.async_copy(src_hbm.at[pl.ds(0, ECH)], sv0, sem_s0)
    pltpu.async_copy(dst_hbm.at[pl.ds(0, ECH)], dv0, sem_d0)
    pltpu.async_copy(src_hbm.at[pl.ds(ECH, ECH)], sv1, sem_s1)
    pltpu.async_copy(dst_hbm.at[pl.ds(ECH, ECH)], dv1, sem_d1)

    for c in range(CPT):
        pltpu.sync_copy(h_hbm.at[pl.ds((c0 + c) * N, N)], hlocs[c])

    sent = jnp.full((L,), SENTINEL, jnp.float32)

    def init_row(i, _):
        for p in range(2):
            for c in range(CPT):
                acc_par[p][c][pl.ds(i * L, L)] = sent
        return 0

    lax.fori_loop(0, N // L, init_row, 0)

    lanes = lax.iota(jnp.int32, L)

    def _vec_has_dup(d):
        srt = lax.sort(d)
        prev = lax.gather(
            srt, jnp.maximum(lanes - 1, 0)[:, None],
            dimension_numbers=lax.GatherDimensionNumbers(
                offset_dims=(), collapsed_slice_dims=(0,),
                start_index_map=(0,)),
            slice_sizes=(1,),
            mode=lax.GatherScatterMode.PROMISE_IN_BOUNDS)
        return jnp.any((srt == prev) & (lanes > 0))

    U = 4

    def process_chunk(svr, dvr):
        def group(g, _):
            base = g * U
            svecs, dvecs, dups = [], [], []
            for u in range(U):
                i = base + u
                s = svr[pl.ds(i * L, L)]
                d = dvr[pl.ds(i * L, L)]
                svecs.append(s)
                dvecs.append(d)
                dups.append(_vec_has_dup(d))
                accs = acc_par[u % 2]
                for c in range(CPT):
                    hv = plsc.load_gather(hlocs[c], [s])
                    cur = plsc.load_gather(accs[c], [d])
                    plsc.store_scatter(accs[c], [d], jnp.minimum(hv, cur))

            any_dup = functools.reduce(jnp.logical_or, dups)

            @pl.when(any_dup)
            def _fixup():
                for u in range(U):
                    accs = acc_par[u % 2]
                    s, d = svecs[u], dvecs[u]

                    @pl.when(dups[u])
                    def _fix_step(s=s, d=d, accs=accs):
                        for c in range(CPT):
                            hv = plsc.load_gather(hlocs[c], [s])
                            chk = plsc.load_gather(accs[c], [d])
                            bad = chk > hv

                            def retry(b, d=d, hv=hv, accs=accs, c=c):
                                cur2 = plsc.load_gather(accs[c], [d])
                                plsc.store_scatter(accs[c], [d],
                                                   jnp.minimum(hv, cur2),
                                                   mask=b)
                                chk2 = plsc.load_gather(accs[c], [d])
                                return b & (chk2 > hv)

                            lax.while_loop(jnp.any, retry, bad)

            return 0

        lax.fori_loop(0, STEPS // U, group, 0)

    def outer(m, _):
        k0 = 2 * m
        pltpu.make_async_copy(src_hbm.at[pl.ds(0, ECH)], sv0, sem_s0).wait()
        pltpu.make_async_copy(dst_hbm.at[pl.ds(0, ECH)], dv0, sem_d0).wait()
        process_chunk(sv0, dv0)

        @pl.when(k0 + 2 < NCHUNK)
        def _prefetch0():
            off = (k0 + 2) * ECH
            pltpu.async_copy(src_hbm.at[pl.ds(off, ECH)], sv0, sem_s0)
            pltpu.async_copy(dst_hbm.at[pl.ds(off, ECH)], dv0, sem_d0)

        pltpu.make_async_copy(src_hbm.at[pl.ds(0, ECH)], sv1, sem_s1).wait()
        pltpu.make_async_copy(dst_hbm.at[pl.ds(0, ECH)], dv1, sem_d1).wait()
        process_chunk(sv1, dv1)

        @pl.when(k0 + 3 < NCHUNK)
        def _prefetch1():
            off = (k0 + 3) * ECH
            pltpu.async_copy(src_hbm.at[pl.ds(off, ECH)], sv1, sem_s1)
            pltpu.async_copy(dst_hbm.at[pl.ds(off, ECH)], dv1, sem_d1)

        return 0

    lax.fori_loop(0, NCHUNK // 2, outer, 0)

    def merge(i, _):
        sl = pl.ds(i * L, L)
        for c in range(CPT):
            acc_par[0][c][sl] = jnp.minimum(acc_par[0][c][sl],
                                            acc_par[1][c][sl])
        return 0

    lax.fori_loop(0, N // L, merge, 0)

    for c in range(CPT):
        pltpu.sync_copy(acc_par[0][c], out_hbm.at[pl.ds((c0 + c) * N, N)])


_segmin = functools.partial(
    pl.kernel,
    out_type=jax.ShapeDtypeStruct((C * N,), jnp.float32),
    mesh=plsc.VectorSubcoreMesh(core_axis_name="c", subcore_axis_name="s",
                                num_cores=NC, num_subcores=NS),
    compiler_params=pltpu.CompilerParams(needs_layout_passes=False),
    scratch_types=(
        [pltpu.VMEM((N,), jnp.float32) for _ in range(CPT)]
        + [pltpu.VMEM((N,), jnp.float32) for _ in range(2 * CPT)]
        + [pltpu.VMEM((ECH,), jnp.int32) for _ in range(4)]
        + [pltpu.SemaphoreType.DMA for _ in range(4)]
    ),
)(_segmin_body)


def _stage_c_body(ht_ref, sm_ref, x_ref, wa_ref, wb_ref, bmr_ref,
                  w2_ref, b2_ref, g2_ref, bb2_ref, o_ref):
    ht = ht_ref[...]
    sm = sm_ref[...]
    md = jnp.where(sm >= 1.5e38, 0.0, ht - sm)
    z = (jnp.dot(wa_ref[...], ht, preferred_element_type=jnp.float32)
         + jnp.dot(wb_ref[...], md, preferred_element_type=jnp.float32)
         + bmr_ref[...])
    z = jnp.maximum(z, 0.0)
    y = jnp.dot(w2_ref[...], z, preferred_element_type=jnp.float32)
    y = y + b2_ref[...]
    m = jnp.mean(y, axis=1, keepdims=True)
    v = jnp.mean((y - m) ** 2, axis=1, keepdims=True)
    y = g2_ref[...] * (y - m) * lax.rsqrt(v + 1e-5) + bb2_ref[...]
    o_ref[...] = jnp.maximum(y + x_ref[...], 0.0)


def _stage_c(ht, smin, x2d, wa, wb, bmr, w2, b2, g2, bb2):
    return pl.pallas_call(
        _stage_c_body,
        out_shape=jax.ShapeDtypeStruct((C, N), jnp.float32),
    )(ht, smin, x2d, wa, wb, bmr, w2, b2, g2, bb2)


def kernel(x, edge_index, W_fc1, b_fc1, bn1_g, bn1_b, W_mr, b_mr,
           W_fc2, b_fc2, bn2_g, bn2_b):
    x2d = x[0]
    src = edge_index[0]
    dst = edge_index[1]

    ht = _stage_a(x2d, W_fc1, b_fc1[:, None], bn1_g[:, None], bn1_b[:, None])
    smin = _segmin(ht.reshape(-1), src, dst).reshape(C, N)
    out = _stage_c(ht, smin, x2d,
                   W_mr[:, :C], W_mr[:, C:], b_mr[:, None],
                   W_fc2, b_fc2[:, None], bn2_g[:, None], bn2_b[:, None])
    return out[None]


# speedup vs baseline: 1.0128x; 1.0128x over previous
import functools

import jax
import jax.numpy as jnp
from jax import lax
from jax.experimental import pallas as pl
from jax.experimental.pallas import tpu as pltpu
from jax.experimental.pallas import tpu_sc as plsc

C = 128
N = 10000
E = 320000

NC = 2
NS = 16
L = 16
NW = NC * NS
CPT = C // NW
ECH = 2000
NCHUNK = E // ECH
STEPS = ECH // L
SENTINEL = 3.0e38


def _fc1_bn_relu_body(x_ref, w_ref, b_ref, g_ref, bb_ref, o_ref):
    z = jnp.dot(w_ref[...], x_ref[...], preferred_element_type=jnp.float32)
    z = z + b_ref[...]
    m = jnp.mean(z, axis=1, keepdims=True)
    v = jnp.mean((z - m) ** 2, axis=1, keepdims=True)
    h = g_ref[...] * (z - m) * lax.rsqrt(v + 1e-5) + bb_ref[...]
    o_ref[...] = jnp.maximum(h, 0.0)


def _stage_a(x2d, w, b, g, bb):
    return pl.pallas_call(
        _fc1_bn_relu_body,
        out_shape=jax.ShapeDtypeStruct((C, N), jnp.float32),
    )(x2d, w, b, g, bb)


def _segmin_body(h_hbm, src_hbm, dst_hbm, out_hbm, *refs):
    hlocs = refs[0:CPT]
    acc_par = (refs[CPT:2 * CPT], refs[2 * CPT:3 * CPT])
    sv0, dv0, sv1, dv1 = refs[3 * CPT:3 * CPT + 4]
    sem_s0, sem_d0, sem_s1, sem_d1 = refs[3 * CPT + 4:3 * CPT + 8]

    cid = lax.axis_index("c")
    sid = lax.axis_index("s")
    wid = sid * NC + cid
    c0 = wid * CPT

    pltpu.async_copy(src_hbm.at[pl.ds(0, ECH)], sv0, sem_s0)
    pltpu.async_copy(dst_hbm.at[pl.ds(0, ECH)], dv0, sem_d0)
    pltpu.async_copy(src_hbm.at[pl.ds(ECH, ECH)], sv1, sem_s1)
    pltpu.async_copy(dst_hbm.at[pl.ds(ECH, ECH)], dv1, sem_d1)

    for c in range(CPT):
        pltpu.sync_copy(h_hbm.at[pl.ds((c0 + c) * N, N)], hlocs[c])

    sent = jnp.full((L,), SENTINEL, jnp.float32)

    def init_row(i, _):
        for p in range(2):
            for c in range(CPT):
                acc_par[p][c][pl.ds(i * L, L)] = sent
        return 0

    lax.fori_loop(0, N // L, init_row, 0)

    lanes = lax.iota(jnp.int32, L)

    def _vec_has_dup(d):
        srt = lax.sort(d)
        prev = lax.gather(
            srt, jnp.maximum(lanes - 1, 0)[:, None],
            dimension_numbers=lax.GatherDimensionNumbers(
                offset_dims=(), collapsed_slice_dims=(0,),
                start_index_map=(0,)),
            slice_sizes=(1,),
            mode=lax.GatherScatterMode.PROMISE_IN_BOUNDS)
        return jnp.any((srt == prev) & (lanes > 0))

    U = 5

    def process_chunk(svr, dvr):
        def group(g, _):
            base = g * U
            svecs, dvecs, dups = [], [], []
            for u in range(U):
                i = base + u
                s = svr[pl.ds(i * L, L)]
                d = dvr[pl.ds(i * L, L)]
                svecs.append(s)
                dvecs.append(d)
                dups.append(_vec_has_dup(d))
                accs = acc_par[u % 2]
                for c in range(CPT):
                    hv = plsc.load_gather(hlocs[c], [s])
                    cur = plsc.load_gather(accs[c], [d])
                    plsc.store_scatter(accs[c], [d], jnp.minimum(hv, cur))

            any_dup = functools.reduce(jnp.logical_or, dups)

            @pl.when(any_dup)
            def _fixup():
                for u in range(U):
                    accs = acc_par[u % 2]
                    s, d = svecs[u], dvecs[u]

                    @pl.when(dups[u])
                    def _fix_step(s=s, d=d, accs=accs):
                        for c in range(CPT):
                            hv = plsc.load_gather(hlocs[c], [s])
                            chk = plsc.load_gather(accs[c], [d])
                            bad = chk > hv

                            def retry(b, d=d, hv=hv, accs=accs, c=c):
                                cur2 = plsc.load_gather(accs[c], [d])
                                plsc.store_scatter(accs[c], [d],
                                                   jnp.minimum(hv, cur2),
                                                   mask=b)
                                chk2 = plsc.load_gather(accs[c], [d])
                                return b & (chk2 > hv)

                            lax.while_loop(jnp.any, retry, bad)

            return 0

        lax.fori_loop(0, STEPS // U, group, 0)

    def outer(m, _):
        k0 = 2 * m
        pltpu.make_async_copy(src_hbm.at[pl.ds(0, ECH)], sv0, sem_s0).wait()
        pltpu.make_async_copy(dst_hbm.at[pl.ds(0, ECH)], dv0, sem_d0).wait()
        process_chunk(sv0, dv0)

        @pl.when(k0 + 2 < NCHUNK)
        def _prefetch0():
            off = (k0 + 2) * ECH
            pltpu.async_copy(src_hbm.at[pl.ds(off, ECH)], sv0, sem_s0)
            pltpu.async_copy(dst_hbm.at[pl.ds(off, ECH)], dv0, sem_d0)

        pltpu.make_async_copy(src_hbm.at[pl.ds(0, ECH)], sv1, sem_s1).wait()
        pltpu.make_async_copy(dst_hbm.at[pl.ds(0, ECH)], dv1, sem_d1).wait()
        process_chunk(sv1, dv1)

        @pl.when(k0 + 3 < NCHUNK)
        def _prefetch1():
            off = (k0 + 3) * ECH
            pltpu.async_copy(src_hbm.at[pl.ds(off, ECH)], sv1, sem_s1)
            pltpu.async_copy(dst_hbm.at[pl.ds(off, ECH)], dv1, sem_d1)

        return 0

    lax.fori_loop(0, NCHUNK // 2, outer, 0)

    def merge(i, _):
        sl = pl.ds(i * L, L)
        for c in range(CPT):
            acc_par[0][c][sl] = jnp.minimum(acc_par[0][c][sl],
                                            acc_par[1][c][sl])
        return 0

    lax.fori_loop(0, N // L, merge, 0)

    for c in range(CPT):
        pltpu.sync_copy(acc_par[0][c], out_hbm.at[pl.ds((c0 + c) * N, N)])


_segmin = functools.partial(
    pl.kernel,
    out_type=jax.ShapeDtypeStruct((C * N,), jnp.float32),
    mesh=plsc.VectorSubcoreMesh(core_axis_name="c", subcore_axis_name="s",
                                num_cores=NC, num_subcores=NS),
    compiler_params=pltpu.CompilerParams(needs_layout_passes=False),
    scratch_types=(
        [pltpu.VMEM((N,), jnp.float32) for _ in range(CPT)]
        + [pltpu.VMEM((N,), jnp.float32) for _ in range(2 * CPT)]
        + [pltpu.VMEM((ECH,), jnp.int32) for _ in range(4)]
        + [pltpu.SemaphoreType.DMA for _ in range(4)]
    ),
)(_segmin_body)


def _stage_c_body(ht_ref, sm_ref, x_ref, wa_ref, wb_ref, bmr_ref,
                  w2_ref, b2_ref, g2_ref, bb2_ref, o_ref):
    ht = ht_ref[...]
    sm = sm_ref[...]
    md = jnp.where(sm >= 1.5e38, 0.0, ht - sm)
    z = (jnp.dot(wa_ref[...], ht, preferred_element_type=jnp.float32)
         + jnp.dot(wb_ref[...], md, preferred_element_type=jnp.float32)
         + bmr_ref[...])
    z = jnp.maximum(z, 0.0)
    y = jnp.dot(w2_ref[...], z, preferred_element_type=jnp.float32)
    y = y + b2_ref[...]
    m = jnp.mean(y, axis=1, keepdims=True)
    v = jnp.mean((y - m) ** 2, axis=1, keepdims=True)
    y = g2_ref[...] * (y - m) * lax.rsqrt(v + 1e-5) + bb2_ref[...]
    o_ref[...] = jnp.maximum(y + x_ref[...], 0.0)


def _stage_c(ht, smin, x2d, wa, wb, bmr, w2, b2, g2, bb2):
    return pl.pallas_call(
        _stage_c_body,
        out_shape=jax.ShapeDtypeStruct((C, N), jnp.float32),
    )(ht, smin, x2d, wa, wb, bmr, w2, b2, g2, bb2)


def kernel(x, edge_index, W_fc1, b_fc1, bn1_g, bn1_b, W_mr, b_mr,
           W_fc2, b_fc2, bn2_g, bn2_b):
    x2d = x[0]
    src = edge_index[0]
    dst = edge_index[1]

    ht = _stage_a(x2d, W_fc1, b_fc1[:, None], bn1_g[:, None], bn1_b[:, None])
    smin = _segmin(ht.reshape(-1), src, dst).reshape(C, N)
    out = _stage_c(ht, smin, x2d,
                   W_mr[:, :C], W_mr[:, C:], b_mr[:, None],
                   W_fc2, b_fc2[:, None], bn2_g[:, None], bn2_b[:, None])
    return out[None]


# speedup vs baseline: 1.0179x; 1.0050x over previous
import functools

import jax
import jax.numpy as jnp
from jax import lax
from jax.experimental import pallas as pl
from jax.experimental.pallas import tpu as pltpu
from jax.experimental.pallas import tpu_sc as plsc

C = 128
N = 10000
E = 320000

NC = 2
NS = 16
L = 16
NW = NC * NS
CPT = C // NW
ECH = 2000
NCHUNK = E // ECH
STEPS = ECH // L
SENTINEL = 3.0e38


def _fc1_bn_relu_body(x_ref, w_ref, b_ref, g_ref, bb_ref, o_ref):
    z = jnp.dot(w_ref[...], x_ref[...], preferred_element_type=jnp.float32)
    z = z + b_ref[...]
    m = jnp.mean(z, axis=1, keepdims=True)
    v = jnp.mean((z - m) ** 2, axis=1, keepdims=True)
    h = g_ref[...] * (z - m) * lax.rsqrt(v + 1e-5) + bb_ref[...]
    o_ref[...] = jnp.maximum(h, 0.0)


def _stage_a(x2d, w, b, g, bb):
    return pl.pallas_call(
        _fc1_bn_relu_body,
        out_shape=jax.ShapeDtypeStruct((C, N), jnp.float32),
    )(x2d, w, b, g, bb)


def _segmin_body(h_hbm, src_hbm, dst_hbm, out_hbm, *refs):
    hlocs = refs[0:CPT]
    accs = refs[CPT:2 * CPT]
    sv0, dv0, sv1, dv1 = refs[2 * CPT:2 * CPT + 4]
    sem_s0, sem_d0, sem_s1, sem_d1 = refs[2 * CPT + 4:2 * CPT + 8]

    cid = lax.axis_index("c")
    sid = lax.axis_index("s")
    wid = sid * NC + cid
    c0 = wid * CPT

    pltpu.async_copy(src_hbm.at[pl.ds(0, ECH)], sv0, sem_s0)
    pltpu.async_copy(dst_hbm.at[pl.ds(0, ECH)], dv0, sem_d0)
    pltpu.async_copy(src_hbm.at[pl.ds(ECH, ECH)], sv1, sem_s1)
    pltpu.async_copy(dst_hbm.at[pl.ds(ECH, ECH)], dv1, sem_d1)

    for c in range(CPT):
        pltpu.sync_copy(h_hbm.at[pl.ds((c0 + c) * N, N)], hlocs[c])

    sent = jnp.full((L,), SENTINEL, jnp.float32)

    def init_row(i, _):
        for c in range(CPT):
            accs[c][pl.ds(i * L, L)] = sent
        return 0

    lax.fori_loop(0, N // L, init_row, 0)

    lanes = lax.iota(jnp.int32, L)

    def _vec_has_dup(d):
        srt = lax.sort(d)
        prev = lax.gather(
            srt, jnp.maximum(lanes - 1, 0)[:, None],
            dimension_numbers=lax.GatherDimensionNumbers(
                offset_dims=(), collapsed_slice_dims=(0,),
                start_index_map=(0,)),
            slice_sizes=(1,),
            mode=lax.GatherScatterMode.PROMISE_IN_BOUNDS)
        return jnp.any((srt == prev) & (lanes > 0))

    U = 5

    def process_chunk(svr, dvr):
        def group(g, _):
            base = g * U
            svecs, dvecs, dups = [], [], []
            for u in range(U):
                i = base + u
                s = svr[pl.ds(i * L, L)]
                d = dvr[pl.ds(i * L, L)]
                svecs.append(s)
                dvecs.append(d)
                dups.append(_vec_has_dup(d))
                for c in range(CPT):
                    hv = plsc.load_gather(hlocs[c], [s])
                    cur = plsc.load_gather(accs[c], [d])
                    plsc.store_scatter(accs[c], [d], jnp.minimum(hv, cur))

            any_dup = functools.reduce(jnp.logical_or, dups)

            @pl.when(any_dup)
            def _fixup():
                for u in range(U):
                    s, d = svecs[u], dvecs[u]

                    @pl.when(dups[u])
                    def _fix_step(s=s, d=d):
                        for c in range(CPT):
                            hv = plsc.load_gather(hlocs[c], [s])
                            chk = plsc.load_gather(accs[c], [d])
                            bad = chk > hv

                            def retry(b, d=d, hv=hv, c=c):
                                cur2 = plsc.load_gather(accs[c], [d])
                                plsc.store_scatter(accs[c], [d],
                                                   jnp.minimum(hv, cur2),
                                                   mask=b)
                                chk2 = plsc.load_gather(accs[c], [d])
                                return b & (chk2 > hv)

                            lax.while_loop(jnp.any, retry, bad)

            return 0

        lax.fori_loop(0, STEPS // U, group, 0)

    def outer(m, _):
        k0 = 2 * m
        pltpu.make_async_copy(src_hbm.at[pl.ds(0, ECH)], sv0, sem_s0).wait()
        pltpu.make_async_copy(dst_hbm.at[pl.ds(0, ECH)], dv0, sem_d0).wait()
        process_chunk(sv0, dv0)

        @pl.when(k0 + 2 < NCHUNK)
        def _prefetch0():
            off = (k0 + 2) * ECH
            pltpu.async_copy(src_hbm.at[pl.ds(off, ECH)], sv0, sem_s0)
            pltpu.async_copy(dst_hbm.at[pl.ds(off, ECH)], dv0, sem_d0)

        pltpu.make_async_copy(src_hbm.at[pl.ds(0, ECH)], sv1, sem_s1).wait()
        pltpu.make_async_copy(dst_hbm.at[pl.ds(0, ECH)], dv1, sem_d1).wait()
        process_chunk(sv1, dv1)

        @pl.when(k0 + 3 < NCHUNK)
        def _prefetch1():
            off = (k0 + 3) * ECH
            pltpu.async_copy(src_hbm.at[pl.ds(off, ECH)], sv1, sem_s1)
            pltpu.async_copy(dst_hbm.at[pl.ds(off, ECH)], dv1, sem_d1)

        return 0

    lax.fori_loop(0, NCHUNK // 2, outer, 0)

    for c in range(CPT):
        pltpu.sync_copy(accs[c], out_hbm.at[pl.ds((c0 + c) * N, N)])


_segmin = functools.partial(
    pl.kernel,
    out_type=jax.ShapeDtypeStruct((C * N,), jnp.float32),
    mesh=plsc.VectorSubcoreMesh(core_axis_name="c", subcore_axis_name="s",
                                num_cores=NC, num_subcores=NS),
    compiler_params=pltpu.CompilerParams(needs_layout_passes=False),
    scratch_types=(
        [pltpu.VMEM((N,), jnp.float32) for _ in range(CPT)]
        + [pltpu.VMEM((N,), jnp.float32) for _ in range(CPT)]
        + [pltpu.VMEM((ECH,), jnp.int32) for _ in range(4)]
        + [pltpu.SemaphoreType.DMA for _ in range(4)]
    ),
)(_segmin_body)


def _stage_c_body(ht_ref, sm_ref, x_ref, wa_ref, wb_ref, bmr_ref,
                  w2_ref, b2_ref, g2_ref, bb2_ref, o_ref):
    ht = ht_ref[...]
    sm = sm_ref[...]
    md = jnp.where(sm >= 1.5e38, 0.0, ht - sm)
    z = (jnp.dot(wa_ref[...], ht, preferred_element_type=jnp.float32)
         + jnp.dot(wb_ref[...], md, preferred_element_type=jnp.float32)
         + bmr_ref[...])
    z = jnp.maximum(z, 0.0)
    y = jnp.dot(w2_ref[...], z, preferred_element_type=jnp.float32)
    y = y + b2_ref[...]
    m = jnp.mean(y, axis=1, keepdims=True)
    v = jnp.mean((y - m) ** 2, axis=1, keepdims=True)
    y = g2_ref[...] * (y - m) * lax.rsqrt(v + 1e-5) + bb2_ref[...]
    o_ref[...] = jnp.maximum(y + x_ref[...], 0.0)


def _stage_c(ht, smin, x2d, wa, wb, bmr, w2, b2, g2, bb2):
    return pl.pallas_call(
        _stage_c_body,
        out_shape=jax.ShapeDtypeStruct((C, N), jnp.float32),
    )(ht, smin, x2d, wa, wb, bmr, w2, b2, g2, bb2)


def kernel(x, edge_index, W_fc1, b_fc1, bn1_g, bn1_b, W_mr, b_mr,
           W_fc2, b_fc2, bn2_g, bn2_b):
    x2d = x[0]
    src = edge_index[0]
    dst = edge_index[1]

    ht = _stage_a(x2d, W_fc1, b_fc1[:, None], bn1_g[:, None], bn1_b[:, None])
    smin = _segmin(ht.reshape(-1), src, dst).reshape(C, N)
    out = _stage_c(ht, smin, x2d,
                   W_mr[:, :C], W_mr[:, C:], b_mr[:, None],
                   W_fc2, b_fc2[:, None], bn2_g[:, None], bn2_b[:, None])
    return out[None]
